# baseline (speedup 1.0000x reference)
import jax
import jax.numpy as jnp
from jax.experimental import pallas as pl
from jax.experimental.pallas import tpu as pltpu

D_MODEL_C = 1024
D_LAT_C = 4096
K_C = 32
ROWS = 2048
BLK = 256


def _body(x_ref, we_ref, b1_ref, b2_ref, wd_ref, lat_ref, rec_ref):
    x = x_ref[...]
    pre = jax.lax.dot_general(
        x, we_ref[...], (((1,), (1,)), ((), ())),
        preferred_element_type=jnp.float32)
    pre = pre + b1_ref[...] + b2_ref[...]

    bits = jax.lax.bitcast_convert_type(pre, jnp.uint32)
    neg = bits >= jnp.uint32(0x80000000)
    key = jnp.where(neg, ~bits, bits | jnp.uint32(0x80000000))

    def cond(state):
        b, _, cnt_at_prefix = state
        return jnp.logical_and(b < 32, jnp.any(cnt_at_prefix != K_C))

    def step(state):
        b, prefix, cnt_at_prefix = state
        bit = jax.lax.shift_left(jnp.uint32(1), jnp.uint32(31) - b.astype(jnp.uint32))
        cand = prefix | bit
        cnt = jnp.sum((key >= cand).astype(jnp.int32), axis=1, keepdims=True)
        take = cnt >= K_C
        return (b + 1,
                jnp.where(take, cand, prefix),
                jnp.where(take, cnt, cnt_at_prefix))

    _, prefix, _ = jax.lax.while_loop(
        cond, step,
        (jnp.int32(0), jnp.zeros((BLK, 1), jnp.uint32),
         jnp.full((BLK, 1), D_LAT_C, jnp.int32)))

    lat = jnp.where(key >= prefix, jnp.maximum(pre, 0.0), 0.0)
    lat_ref[...] = lat
    rec_ref[...] = jax.lax.dot_general(
        lat.astype(jnp.bfloat16), wd_ref[...].astype(jnp.bfloat16),
        (((1,), (1,)), ((), ())),
        preferred_element_type=jnp.float32)


def kernel(x, W_enc, b_enc, enc_bias, W_dec):
    B, S, D = x.shape
    x2 = x.reshape(B * S, D)
    b1 = b_enc.reshape(1, D_LAT_C)
    b2 = enc_bias.reshape(1, D_LAT_C)
    grid = (B * S) // BLK

    lat2, rec2 = pl.pallas_call(
        _body,
        grid=(grid,),
        in_specs=[
            pl.BlockSpec((BLK, D), lambda i: (i, 0)),
            pl.BlockSpec((D_LAT_C, D), lambda i: (0, 0)),
            pl.BlockSpec((1, D_LAT_C), lambda i: (0, 0)),
            pl.BlockSpec((1, D_LAT_C), lambda i: (0, 0)),
            pl.BlockSpec((D, D_LAT_C), lambda i: (0, 0)),
        ],
        out_specs=[
            pl.BlockSpec((BLK, D_LAT_C), lambda i: (i, 0)),
            pl.BlockSpec((BLK, D), lambda i: (i, 0)),
        ],
        out_shape=[
            jax.ShapeDtypeStruct((B * S, D_LAT_C), jnp.float32),
            jax.ShapeDtypeStruct((B * S, D), jnp.float32),
        ],
        compiler_params=pltpu.CompilerParams(
            dimension_semantics=("arbitrary",),
        ),
    )(x2, W_enc, b1, b2, W_dec)

    return rec2.reshape(B, S, D), lat2.reshape(B, S, D_LAT_C)


# speedup vs baseline: 1.1679x; 1.1679x over previous
import jax
import jax.numpy as jnp
from jax.experimental import pallas as pl
from jax.experimental.pallas import tpu as pltpu

D_MODEL_C = 1024
D_LAT_C = 4096
K_C = 32
ROWS = 2048
BLK = 256


def _body(x_ref, we_ref, b1_ref, b2_ref, wd_ref, lat_ref, rec_ref):
    x = x_ref[...]
    pre = jax.lax.dot_general(
        x, we_ref[...], (((1,), (1,)), ((), ())),
        preferred_element_type=jnp.float32)
    pre = pre + b1_ref[...] + b2_ref[...]

    def f2key(v):
        bits = jax.lax.bitcast_convert_type(v, jnp.uint32)
        neg = bits >= jnp.uint32(0x80000000)
        return jnp.where(neg, ~bits, bits | jnp.uint32(0x80000000))

    key = f2key(pre)

    ch = jnp.max(pre.reshape(BLK, 32, 128), axis=2)
    tau0 = jnp.min(ch, axis=1, keepdims=True)
    rmax = jnp.max(ch, axis=1, keepdims=True)
    k0 = f2key(tau0)
    k0 = jnp.where(k0 >= jnp.uint32(1), k0 - jnp.uint32(1), jnp.uint32(0))
    kM = f2key(rmax)
    rng = kM - k0
    e = (jax.lax.bitcast_convert_type(rng.astype(jnp.float32), jnp.int32)
         >> 23) - 127
    shift = jnp.clip(31 - e, 0, 31).astype(jnp.uint32)
    keyn = jnp.where(key > k0,
                     jax.lax.shift_left(key - k0, shift),
                     jnp.uint32(0))

    def step(b, carry):
        prefix, cnt_at = carry
        bit = jax.lax.shift_left(jnp.uint32(1), jnp.uint32(31) - b.astype(jnp.uint32))
        cand = prefix | bit
        cnt = jnp.sum((keyn >= cand).astype(jnp.int32), axis=1, keepdims=True)
        take = cnt >= K_C
        return (jnp.where(take, cand, prefix), jnp.where(take, cnt, cnt_at))

    carry0 = (jnp.zeros((BLK, 1), jnp.uint32),
              jnp.full((BLK, 1), D_LAT_C, jnp.int32))
    carry = jax.lax.fori_loop(0, 16, step, carry0, unroll=True)

    def finish(c):
        return jax.lax.fori_loop(16, 32, step, c, unroll=True)

    prefix, _ = jax.lax.cond(
        jnp.any(carry[1] != K_C), finish, lambda c: c, carry)

    lat = jnp.where(keyn >= prefix, jnp.maximum(pre, 0.0), 0.0)
    lat_ref[...] = lat
    rec_ref[...] = jax.lax.dot_general(
        lat.astype(jnp.bfloat16), wd_ref[...].astype(jnp.bfloat16),
        (((1,), (1,)), ((), ())),
        preferred_element_type=jnp.float32)


def kernel(x, W_enc, b_enc, enc_bias, W_dec):
    B, S, D = x.shape
    x2 = x.reshape(B * S, D)
    b1 = b_enc.reshape(1, D_LAT_C)
    b2 = enc_bias.reshape(1, D_LAT_C)
    grid = (B * S) // BLK

    lat2, rec2 = pl.pallas_call(
        _body,
        grid=(grid,),
        in_specs=[
            pl.BlockSpec((BLK, D), lambda i: (i, 0)),
            pl.BlockSpec((D_LAT_C, D), lambda i: (0, 0)),
            pl.BlockSpec((1, D_LAT_C), lambda i: (0, 0)),
            pl.BlockSpec((1, D_LAT_C), lambda i: (0, 0)),
            pl.BlockSpec((D, D_LAT_C), lambda i: (0, 0)),
        ],
        out_specs=[
            pl.BlockSpec((BLK, D_LAT_C), lambda i: (i, 0)),
            pl.BlockSpec((BLK, D), lambda i: (i, 0)),
        ],
        out_shape=[
            jax.ShapeDtypeStruct((B * S, D_LAT_C), jnp.float32),
            jax.ShapeDtypeStruct((B * S, D), jnp.float32),
        ],
        compiler_params=pltpu.CompilerParams(
            dimension_semantics=("arbitrary",),
        ),
    )(x2, W_enc, b1, b2, W_dec)

    return rec2.reshape(B, S, D), lat2.reshape(B, S, D_LAT_C)
